# Initial kernel scaffold; baseline (speedup 1.0000x reference)
import jax
import jax.numpy as jnp
from jax.experimental import pallas as pl


def kernel(x, edge_index, W1, b1, W2, b2):
    raise NotImplementedError("write your pallas kernel here")


# speedup vs baseline: 8.6638x; 8.6638x over previous
import functools

import jax
import jax.numpy as jnp
from jax import lax
from jax.experimental import pallas as pl
from jax.experimental.pallas import tpu as pltpu
from jax.experimental.pallas import tpu_sc as plsc

N = 10000
NR = 10240
E = 320000
ROWS = 2560
PE = ROWS * 128
DUMMY = N
STRIPE = NR // 16

_MESH = plsc.VectorSubcoreMesh(core_axis_name="c", subcore_axis_name="s")


@functools.partial(
    pl.kernel,
    out_type=jax.ShapeDtypeStruct((2 * NR,), jnp.float32),
    mesh=_MESH,
    scratch_types=[
        pltpu.VMEM((80, 128), jnp.int32),
        pltpu.VMEM((128,), jnp.float32),
        pltpu.VMEM_SHARED((NR,), jnp.float32),
    ],
)
def _sc_degree(dst_hbm, zeros1_hbm, out_hbm, dst_v, ones_v, acc_sh):
    c = lax.axis_index("c")
    s = lax.axis_index("s")
    wid = c * 16 + s
    for i in range(8):
        ones_v[pl.ds(i * 16, 16)] = jnp.ones((16,), jnp.float32)
    pltpu.sync_copy(dst_hbm.at[pl.ds(wid * 80, 80)], dst_v)
    pltpu.sync_copy(zeros1_hbm.at[pl.ds(s * STRIPE, STRIPE)],
                    acc_sh.at[pl.ds(s * STRIPE, STRIPE)])
    plsc.subcore_barrier()

    def body(j, carry):
        pltpu.sync_copy(ones_v, acc_sh.at[dst_v.at[j]], add=True)
        return carry

    lax.fori_loop(0, 80, body, 0)
    plsc.subcore_barrier()
    pltpu.sync_copy(acc_sh.at[pl.ds(s * STRIPE, STRIPE)],
                    out_hbm.at[pl.ds(c * NR + s * STRIPE, STRIPE)])


_SEG = 80


def _make_sc_scatter(feature_split: bool):
    cpt = (2 * ROWS) // 32 if feature_split else ROWS // 32
    nseg = cpt // _SEG

    @functools.partial(
        pl.kernel,
        out_type=jax.ShapeDtypeStruct((2 * NR, 128), jnp.float32),
        mesh=_MESH,
        scratch_types=[
            pltpu.VMEM((_SEG, 128), jnp.int32),
            pltpu.VMEM((_SEG, 128), jnp.int32),
            pltpu.VMEM((128, 128), jnp.float32),
            pltpu.VMEM_SHARED((NR, 128), jnp.float32),
            pltpu.SemaphoreType.DMA,
        ],
    )
    def _sc_scatter(y_hbm, srcs_hbm, dsts_hbm, zeros_hbm, out_hbm,
                    src_v, dst_v, rows_v, acc_sh, sem):
        c = lax.axis_index("c")
        s = lax.axis_index("s")
        if feature_split:
            src_base = c * ROWS + s * cpt
            dst_base = s * cpt
        else:
            src_base = (c * 16 + s) * cpt
            dst_base = src_base
        pltpu.sync_copy(zeros_hbm.at[pl.ds(s * STRIPE, STRIPE)],
                        acc_sh.at[pl.ds(s * STRIPE, STRIPE)])
        plsc.subcore_barrier()

        def seg_body(g, carry):
            pltpu.sync_copy(srcs_hbm.at[pl.ds(src_base + g * _SEG, _SEG)], src_v)
            pltpu.sync_copy(dsts_hbm.at[pl.ds(dst_base + g * _SEG, _SEG)], dst_v)

            def body(j, carry2):
                pltpu.async_copy(y_hbm.at[src_v.at[j]], rows_v, sem).wait()
                pltpu.sync_copy(rows_v, acc_sh.at[dst_v.at[j]], add=True)
                return carry2

            lax.fori_loop(0, _SEG, body, 0)
            return carry

        lax.fori_loop(0, nseg, seg_body, 0)
        plsc.subcore_barrier()
        pltpu.sync_copy(acc_sh.at[pl.ds(s * STRIPE, STRIPE)],
                        out_hbm.at[pl.ds(c * NR + s * STRIPE, STRIPE)])

    return _sc_scatter


_sc_scatter1 = _make_sc_scatter(feature_split=True)
_sc_scatter2 = _make_sc_scatter(feature_split=False)


_R = 256
_G = NR // _R


def _tc_a_body(x_ref, w_ref, degp_ref, y_ref, dinv_ref):
    deg = degp_ref[0] + degp_ref[1] + 1.0
    dinv = lax.rsqrt(deg)
    xw = jnp.dot(x_ref[...], w_ref[...], preferred_element_type=jnp.float32)
    y = xw * dinv[:, None]
    y_ref[0] = y[:, :128]
    y_ref[1] = y[:, 128:]
    dinv_ref[...] = dinv


_tc_a = pl.pallas_call(
    _tc_a_body,
    grid=(_G,),
    in_specs=[
        pl.BlockSpec((_R, 128), lambda i: (i, 0)),
        pl.BlockSpec((128, 256), lambda i: (0, 0)),
        pl.BlockSpec((2, _R), lambda i: (0, i)),
    ],
    out_specs=[
        pl.BlockSpec((2, _R, 128), lambda i: (0, i, 0)),
        pl.BlockSpec((_R,), lambda i: (i,)),
    ],
    out_shape=[
        jax.ShapeDtypeStruct((2, NR, 128), jnp.float32),
        jax.ShapeDtypeStruct((NR,), jnp.float32),
    ],
)


def _tc_b_body(s1_ref, y1_ref, dinv_ref, b1_ref, w2_ref, y2_ref):
    dinv = dinv_ref[...]
    b1 = b1_ref[...]
    h1a = jnp.maximum(dinv[:, None] * (s1_ref[0] + y1_ref[0]) + b1[:128], 0.0)
    h1b = jnp.maximum(dinv[:, None] * (s1_ref[1] + y1_ref[1]) + b1[128:], 0.0)
    w2 = w2_ref[...]
    y2 = jnp.dot(h1a, w2[:128], preferred_element_type=jnp.float32)
    y2 = y2 + jnp.dot(h1b, w2[128:], preferred_element_type=jnp.float32)
    y2_ref[...] = y2 * dinv[:, None]


_tc_b = pl.pallas_call(
    _tc_b_body,
    grid=(_G,),
    in_specs=[
        pl.BlockSpec((2, _R, 128), lambda i: (0, i, 0)),
        pl.BlockSpec((2, _R, 128), lambda i: (0, i, 0)),
        pl.BlockSpec((_R,), lambda i: (i,)),
        pl.BlockSpec((256,), lambda i: (0,)),
        pl.BlockSpec((256, 128), lambda i: (0, 0)),
    ],
    out_specs=pl.BlockSpec((_R, 128), lambda i: (i, 0)),
    out_shape=jax.ShapeDtypeStruct((NR, 128), jnp.float32),
)


def _tc_c_body(s2_ref, y2_ref, dinv_ref, b2_ref, h_ref):
    dinv = dinv_ref[...]
    acc = s2_ref[0] + s2_ref[1] + y2_ref[...]
    h_ref[...] = jnp.maximum(dinv[:, None] * acc + b2_ref[...][None, :], 0.0)


_tc_c = pl.pallas_call(
    _tc_c_body,
    grid=(_G,),
    in_specs=[
        pl.BlockSpec((2, _R, 128), lambda i: (0, i, 0)),
        pl.BlockSpec((_R, 128), lambda i: (i, 0)),
        pl.BlockSpec((_R,), lambda i: (i,)),
        pl.BlockSpec((128,), lambda i: (0,)),
    ],
    out_specs=pl.BlockSpec((_R, 128), lambda i: (i, 0)),
    out_shape=jax.ShapeDtypeStruct((NR, 128), jnp.float32),
)


def kernel(x, edge_index, W1, b1, W2, b2):
    src = edge_index[0].astype(jnp.int32)
    dst = edge_index[1].astype(jnp.int32)
    pad = PE - E
    srcp = jnp.concatenate([src, jnp.zeros((pad,), jnp.int32)]).reshape(ROWS, 128)
    dstp = jnp.concatenate([dst, jnp.full((pad,), DUMMY, jnp.int32)]).reshape(ROWS, 128)
    srcs2 = jnp.concatenate([srcp, srcp + NR], axis=0)
    z1 = jnp.zeros((NR,), jnp.float32)
    z2 = jnp.zeros((NR, 128), jnp.float32)
    xp = jnp.pad(x, ((0, NR - N), (0, 0)))

    degp = _sc_degree(dstp, z1).reshape(2, NR)
    y1p, dinv = _tc_a(xp, W1, degp)
    s1 = _sc_scatter1(y1p.reshape(2 * NR, 128), srcs2, dstp, z2).reshape(2, NR, 128)
    y2 = _tc_b(s1, y1p, dinv, b1, W2)
    s2 = _sc_scatter2(y2, srcp, dstp, z2).reshape(2, NR, 128)
    h2 = _tc_c(s2, y2, dinv, b2)
    return h2[:N]
